# baseline (speedup 1.0000x reference)
import jax
import jax.numpy as jnp
from jax import lax
from jax.experimental import pallas as pl
from jax.experimental.pallas import tpu as pltpu
from jax.experimental.pallas import tpu_sc as plsc

B = 64
H = 2048
D = 128
NC = 2
NS = 16
NW = NC * NS
BPW = B // NW
E = BPW * H
GCH = 128
CPB = H // GCH


def _sc_body(pred_hbm, targ_hbm, out_hbm, idx_v, col_v, row_v, res_v, all_v,
             shared_v, semc0, semc1, sem2):
    cid = lax.axis_index("c")
    sid = lax.axis_index("s")
    wid = cid * NS + sid
    b0 = wid * BPW
    lanes = lax.iota(jnp.int32, 16)
    lanesD = lanes * D

    base = b0 * H * D + 1
    csems = [semc0, semc1]

    for k in range(BPW):

        def chunk_fire(c, _):
            off = k * H + c * GCH
            for u in range(GCH // 16):
                idx_v[pl.ds(off + u * 16, 16)] = (
                    (base + (off + u * 16) * D) + lanesD)
            pltpu.async_copy(
                targ_hbm.at[idx_v.at[pl.ds(off, GCH)]],
                col_v.at[pl.ds(off, GCH)], csems[k])
            return 0

        lax.fori_loop(0, CPB, chunk_fire, 0)

    def count_batch(k):
        pltpu.make_async_copy(
            targ_hbm.at[idx_v.at[pl.ds(k * H, H)]],
            col_v.at[pl.ds(k * H, H)], csems[k]).wait()

        def cnt(c, acc):
            off = k * H + c * 128
            for u in range(8):
                v = col_v[pl.ds(off + u * 16, 16)]
                acc = acc + jnp.where(v != 0.0, 1, 0).astype(jnp.int32)
            return acc

        accv = lax.fori_loop(0, H // 128, cnt, jnp.zeros((16,), jnp.int32))
        return jnp.sum(accv)

    ts = []
    rs = []
    for k in range(BPW):
        t = count_batch(k)
        r = jnp.where(t == 0, H - 1, t - 1)
        rp = jnp.maximum(r - 1, 0)
        bb = b0 + k
        g = (bb * H + r) * D
        gp = (bb * H + rp) * D
        o = k * 4 * D
        pltpu.async_copy(pred_hbm.at[pl.ds(g, D)], row_v.at[pl.ds(o, D)], sem2)
        pltpu.async_copy(targ_hbm.at[pl.ds(g, D)],
                         row_v.at[pl.ds(o + D, D)], sem2)
        pltpu.async_copy(pred_hbm.at[pl.ds(gp, D)],
                         row_v.at[pl.ds(o + 2 * D, D)], sem2)
        pltpu.async_copy(targ_hbm.at[pl.ds(gp, D)],
                         row_v.at[pl.ds(o + 3 * D, D)], sem2)
        ts.append(t)
        rs.append(r)

    for k in range(BPW):
        o = k * 4 * D
        g = (b0 + k) * H * D
        for j in range(4):
            pltpu.make_async_copy(pred_hbm.at[pl.ds(g, D)],
                                  row_v.at[pl.ds(o + j * D, D)], sem2).wait()

    total = jnp.float32(0.0)
    for k in range(BPW):
        t, r = ts[k], rs[k]
        x = t.astype(jnp.float32) * jnp.float32(1.0 / (H - 1))
        xv = jnp.full((16,), x, jnp.float32)
        iv = plsc.bitcast(xv, jnp.int32)
        yv = plsc.bitcast(jnp.int32(0x5F3759DF) - (iv >> 1), jnp.float32)
        for _ in range(3):
            yv = yv * (1.5 - 0.5 * xv * yv * yv)
        pwv = xv * xv * (xv * yv)
        pw = jnp.sum(jnp.where(lanes == 0, pwv, 0.0))
        pw = jnp.where(t == 0, 0.0, pw)
        wgt = 1.0 + 0.5 * pw

        pm = jnp.where(r > 0, 1.0, 0.0)
        o = k * 4 * D
        z = jnp.zeros((16,), jnp.float32)
        sq, l1 = z, z
        for c in range(D // 16):
            dd = (row_v[pl.ds(o + c * 16, 16)]
                  - row_v[pl.ds(o + D + c * 16, 16)])
            dp = (row_v[pl.ds(o + 2 * D + c * 16, 16)]
                  - row_v[pl.ds(o + 3 * D + c * 16, 16)]) * pm
            sq = sq + dd * dd
            l1 = l1 + jnp.abs(dd - dp)
        total = total + wgt * (jnp.sum(sq) + jnp.sum(l1))

    res_v[...] = jnp.where(lanes == 0, jnp.full((16,), total, jnp.float32),
                           jnp.zeros((16,), jnp.float32))
    pltpu.sync_copy(res_v, shared_v.at[pl.ds(sid * 16, 16)])
    plsc.subcore_barrier()

    @pl.when(sid == 0)
    def _():
        pltpu.sync_copy(shared_v, all_v)
        acc = jnp.zeros((16,), jnp.float32)
        for i in range(NS):
            acc = acc + all_v[pl.ds(i * 16, 16)]
        core_total = jnp.sum(acc)
        res_v[...] = jnp.where(lanes == 0,
                               jnp.full((16,), core_total, jnp.float32),
                               jnp.zeros((16,), jnp.float32))
        pltpu.sync_copy(res_v, out_hbm.at[cid])


@jax.jit
def _sc_loss(pred_flat, targ_flat):
    mesh = plsc.VectorSubcoreMesh(core_axis_name="c", subcore_axis_name="s")
    f = pl.kernel(
        _sc_body,
        out_type=jax.ShapeDtypeStruct((NC, 16), jnp.float32),
        mesh=mesh,
        compiler_params=pltpu.CompilerParams(needs_layout_passes=False),
        scratch_types=[
            pltpu.VMEM((E,), jnp.int32),
            pltpu.VMEM((E,), jnp.float32),
            pltpu.VMEM((4 * BPW * D,), jnp.float32),
            pltpu.VMEM((16,), jnp.float32),
            pltpu.VMEM((NS * 16,), jnp.float32),
            pltpu.VMEM_SHARED((NS * 16,), jnp.float32),
            pltpu.SemaphoreType.DMA,
            pltpu.SemaphoreType.DMA,
            pltpu.SemaphoreType.DMA,
        ],
    )
    return f(pred_flat, targ_flat)


def kernel(pred, targ):
    out = _sc_loss(pred.reshape(-1), targ.reshape(-1))
    loss = (out[0, 0] + out[1, 0]) * (1.0 / B)
    return (loss, {"a0_loss": loss})


# speedup vs baseline: 1.0369x; 1.0369x over previous
import jax
import jax.numpy as jnp
from jax import lax
from jax.experimental import pallas as pl
from jax.experimental.pallas import tpu as pltpu
from jax.experimental.pallas import tpu_sc as plsc

B = 64
H = 2048
D = 128
NC = 2
NS = 16
NW = NC * NS
BPW = B // NW
E = BPW * H
GCH = 128
CPB = H // GCH


def _sc_body(pred_hbm, targ_hbm, out_hbm, idx_v, col_v, row_v, res_v,
             semc0, semc1, sem2):
    cid = lax.axis_index("c")
    sid = lax.axis_index("s")
    wid = cid * NS + sid
    b0 = wid * BPW
    lanes = lax.iota(jnp.int32, 16)
    lanesD = lanes * D

    base = b0 * H * D + 1
    csems = [semc0, semc1]

    for k in range(BPW):

        def chunk_fire(c, _):
            off = k * H + c * GCH
            for u in range(GCH // 16):
                idx_v[pl.ds(off + u * 16, 16)] = (
                    (base + (off + u * 16) * D) + lanesD)
            pltpu.async_copy(
                targ_hbm.at[idx_v.at[pl.ds(off, GCH)]],
                col_v.at[pl.ds(off, GCH)], csems[k])
            return 0

        lax.fori_loop(0, CPB, chunk_fire, 0)

    def count_batch(k):
        pltpu.make_async_copy(
            targ_hbm.at[idx_v.at[pl.ds(k * H, H)]],
            col_v.at[pl.ds(k * H, H)], csems[k]).wait()

        def cnt(c, acc):
            off = k * H + c * 128
            for u in range(8):
                v = col_v[pl.ds(off + u * 16, 16)]
                acc = acc + jnp.where(v != 0.0, 1, 0).astype(jnp.int32)
            return acc

        accv = lax.fori_loop(0, H // 128, cnt, jnp.zeros((16,), jnp.int32))
        return jnp.sum(accv)

    for k in range(BPW):
        bb = b0 + k
        o = k * 4 * D
        g = (bb * H + (H - 1)) * D
        gp = (bb * H + (H - 2)) * D
        pltpu.async_copy(pred_hbm.at[pl.ds(g, D)], row_v.at[pl.ds(o, D)], sem2)
        pltpu.async_copy(targ_hbm.at[pl.ds(g, D)],
                         row_v.at[pl.ds(o + D, D)], sem2)
        pltpu.async_copy(pred_hbm.at[pl.ds(gp, D)],
                         row_v.at[pl.ds(o + 2 * D, D)], sem2)
        pltpu.async_copy(targ_hbm.at[pl.ds(gp, D)],
                         row_v.at[pl.ds(o + 3 * D, D)], sem2)

    ts = []
    rs = []
    for k in range(BPW):
        t = count_batch(k)
        r = jnp.where(t == 0, H - 1, t - 1)
        ts.append(t)
        rs.append(r)

    for k in range(BPW):
        o = k * 4 * D
        g = (b0 + k) * H * D
        for j in range(4):
            pltpu.make_async_copy(pred_hbm.at[pl.ds(g, D)],
                                  row_v.at[pl.ds(o + j * D, D)], sem2).wait()

    for k in range(BPW):
        r = rs[k]

        @pl.when(r != H - 1)
        def _(k=k, r=r):
            rp = jnp.maximum(r - 1, 0)
            bb = b0 + k
            g = (bb * H + r) * D
            gp = (bb * H + rp) * D
            o = k * 4 * D
            pltpu.sync_copy(pred_hbm.at[pl.ds(g, D)], row_v.at[pl.ds(o, D)])
            pltpu.sync_copy(targ_hbm.at[pl.ds(g, D)],
                            row_v.at[pl.ds(o + D, D)])
            pltpu.sync_copy(pred_hbm.at[pl.ds(gp, D)],
                            row_v.at[pl.ds(o + 2 * D, D)])
            pltpu.sync_copy(targ_hbm.at[pl.ds(gp, D)],
                            row_v.at[pl.ds(o + 3 * D, D)])

    total = jnp.float32(0.0)
    for k in range(BPW):
        t, r = ts[k], rs[k]
        x = t.astype(jnp.float32) * jnp.float32(1.0 / (H - 1))
        xv = jnp.full((16,), x, jnp.float32)
        iv = plsc.bitcast(xv, jnp.int32)
        yv = plsc.bitcast(jnp.int32(0x5F3759DF) - (iv >> 1), jnp.float32)
        for _ in range(3):
            yv = yv * (1.5 - 0.5 * xv * yv * yv)
        pwv = xv * xv * (xv * yv)
        pw = jnp.sum(jnp.where(lanes == 0, pwv, 0.0))
        pw = jnp.where(t == 0, 0.0, pw)
        wgt = 1.0 + 0.5 * pw

        pm = jnp.where(r > 0, 1.0, 0.0)
        o = k * 4 * D
        z = jnp.zeros((16,), jnp.float32)
        sq, l1 = z, z
        for c in range(D // 16):
            dd = (row_v[pl.ds(o + c * 16, 16)]
                  - row_v[pl.ds(o + D + c * 16, 16)])
            dp = (row_v[pl.ds(o + 2 * D + c * 16, 16)]
                  - row_v[pl.ds(o + 3 * D + c * 16, 16)]) * pm
            sq = sq + dd * dd
            l1 = l1 + jnp.abs(dd - dp)
        total = total + wgt * (jnp.sum(sq) + jnp.sum(l1))

    res_v[...] = jnp.where(lanes == 0, jnp.full((16,), total, jnp.float32),
                           jnp.zeros((16,), jnp.float32))
    pltpu.sync_copy(res_v, out_hbm.at[wid])


@jax.jit
def _sc_loss(pred_flat, targ_flat):
    mesh = plsc.VectorSubcoreMesh(core_axis_name="c", subcore_axis_name="s")
    f = pl.kernel(
        _sc_body,
        out_type=jax.ShapeDtypeStruct((NW, 16), jnp.float32),
        mesh=mesh,
        compiler_params=pltpu.CompilerParams(needs_layout_passes=False),
        scratch_types=[
            pltpu.VMEM((E,), jnp.int32),
            pltpu.VMEM((E,), jnp.float32),
            pltpu.VMEM((4 * BPW * D,), jnp.float32),
            pltpu.VMEM((16,), jnp.float32),
            pltpu.SemaphoreType.DMA,
            pltpu.SemaphoreType.DMA,
            pltpu.SemaphoreType.DMA,
        ],
    )
    return f(pred_flat, targ_flat)


def kernel(pred, targ):
    out = _sc_loss(pred.reshape(-1), targ.reshape(-1))
    loss = jnp.sum(out) * (1.0 / B)
    return (loss, {"a0_loss": loss})


# speedup vs baseline: 1.0405x; 1.0035x over previous
import jax
import jax.numpy as jnp
from jax import lax
from jax.experimental import pallas as pl
from jax.experimental.pallas import tpu as pltpu
from jax.experimental.pallas import tpu_sc as plsc

B = 64
H = 2048
D = 128
NC = 2
NS = 16
NW = NC * NS
BPW = B // NW
E = BPW * H
GCH = 256
CPB = H // GCH


def _sc_body(pred_hbm, targ_hbm, out_hbm, idx_v, col_v, row_v, res_v,
             semc0, semc1, sem2):
    cid = lax.axis_index("c")
    sid = lax.axis_index("s")
    wid = cid * NS + sid
    b0 = wid * BPW
    lanes = lax.iota(jnp.int32, 16)
    lanesD = lanes * D

    base = b0 * H * D + 1
    csems = [semc0, semc1]

    for k in range(BPW):

        def chunk_fire(c, _):
            off = k * H + c * GCH
            for u in range(GCH // 16):
                idx_v[pl.ds(off + u * 16, 16)] = (
                    (base + (off + u * 16) * D) + lanesD)
            pltpu.async_copy(
                targ_hbm.at[idx_v.at[pl.ds(off, GCH)]],
                col_v.at[pl.ds(off, GCH)], csems[k])
            return 0

        lax.fori_loop(0, CPB, chunk_fire, 0)

    def count_batch(k):
        pltpu.make_async_copy(
            targ_hbm.at[idx_v.at[pl.ds(k * H, H)]],
            col_v.at[pl.ds(k * H, H)], csems[k]).wait()

        def cnt(c, acc):
            off = k * H + c * 128
            for u in range(8):
                v = col_v[pl.ds(off + u * 16, 16)]
                acc = acc + jnp.where(v != 0.0, 1, 0).astype(jnp.int32)
            return acc

        accv = lax.fori_loop(0, H // 128, cnt, jnp.zeros((16,), jnp.int32))
        return jnp.sum(accv)

    for k in range(BPW):
        bb = b0 + k
        o = k * 4 * D
        g = (bb * H + (H - 1)) * D
        gp = (bb * H + (H - 2)) * D
        pltpu.async_copy(pred_hbm.at[pl.ds(g, D)], row_v.at[pl.ds(o, D)], sem2)
        pltpu.async_copy(targ_hbm.at[pl.ds(g, D)],
                         row_v.at[pl.ds(o + D, D)], sem2)
        pltpu.async_copy(pred_hbm.at[pl.ds(gp, D)],
                         row_v.at[pl.ds(o + 2 * D, D)], sem2)
        pltpu.async_copy(targ_hbm.at[pl.ds(gp, D)],
                         row_v.at[pl.ds(o + 3 * D, D)], sem2)

    ts = []
    rs = []
    for k in range(BPW):
        t = count_batch(k)
        r = jnp.where(t == 0, H - 1, t - 1)
        ts.append(t)
        rs.append(r)

    for k in range(BPW):
        o = k * 4 * D
        g = (b0 + k) * H * D
        for j in range(4):
            pltpu.make_async_copy(pred_hbm.at[pl.ds(g, D)],
                                  row_v.at[pl.ds(o + j * D, D)], sem2).wait()

    for k in range(BPW):
        r = rs[k]

        @pl.when(r != H - 1)
        def _(k=k, r=r):
            rp = jnp.maximum(r - 1, 0)
            bb = b0 + k
            g = (bb * H + r) * D
            gp = (bb * H + rp) * D
            o = k * 4 * D
            pltpu.sync_copy(pred_hbm.at[pl.ds(g, D)], row_v.at[pl.ds(o, D)])
            pltpu.sync_copy(targ_hbm.at[pl.ds(g, D)],
                            row_v.at[pl.ds(o + D, D)])
            pltpu.sync_copy(pred_hbm.at[pl.ds(gp, D)],
                            row_v.at[pl.ds(o + 2 * D, D)])
            pltpu.sync_copy(targ_hbm.at[pl.ds(gp, D)],
                            row_v.at[pl.ds(o + 3 * D, D)])

    total = jnp.float32(0.0)
    for k in range(BPW):
        t, r = ts[k], rs[k]
        x = t.astype(jnp.float32) * jnp.float32(1.0 / (H - 1))
        xv = jnp.full((16,), x, jnp.float32)
        iv = plsc.bitcast(xv, jnp.int32)
        yv = plsc.bitcast(jnp.int32(0x5F3759DF) - (iv >> 1), jnp.float32)
        for _ in range(3):
            yv = yv * (1.5 - 0.5 * xv * yv * yv)
        pwv = xv * xv * (xv * yv)
        pw = jnp.sum(jnp.where(lanes == 0, pwv, 0.0))
        pw = jnp.where(t == 0, 0.0, pw)
        wgt = 1.0 + 0.5 * pw

        pm = jnp.where(r > 0, 1.0, 0.0)
        o = k * 4 * D
        z = jnp.zeros((16,), jnp.float32)
        sq, l1 = z, z
        for c in range(D // 16):
            dd = (row_v[pl.ds(o + c * 16, 16)]
                  - row_v[pl.ds(o + D + c * 16, 16)])
            dp = (row_v[pl.ds(o + 2 * D + c * 16, 16)]
                  - row_v[pl.ds(o + 3 * D + c * 16, 16)]) * pm
            sq = sq + dd * dd
            l1 = l1 + jnp.abs(dd - dp)
        total = total + wgt * (jnp.sum(sq) + jnp.sum(l1))

    res_v[...] = jnp.where(lanes == 0, jnp.full((16,), total, jnp.float32),
                           jnp.zeros((16,), jnp.float32))
    pltpu.sync_copy(res_v, out_hbm.at[wid])


@jax.jit
def _sc_loss(pred_flat, targ_flat):
    mesh = plsc.VectorSubcoreMesh(core_axis_name="c", subcore_axis_name="s")
    f = pl.kernel(
        _sc_body,
        out_type=jax.ShapeDtypeStruct((NW, 16), jnp.float32),
        mesh=mesh,
        compiler_params=pltpu.CompilerParams(needs_layout_passes=False),
        scratch_types=[
            pltpu.VMEM((E,), jnp.int32),
            pltpu.VMEM((E,), jnp.float32),
            pltpu.VMEM((4 * BPW * D,), jnp.float32),
            pltpu.VMEM((16,), jnp.float32),
            pltpu.SemaphoreType.DMA,
            pltpu.SemaphoreType.DMA,
            pltpu.SemaphoreType.DMA,
        ],
    )
    return f(pred_flat, targ_flat)


def kernel(pred, targ):
    out = _sc_loss(pred.reshape(-1), targ.reshape(-1))
    loss = jnp.sum(out) * (1.0 / B)
    return (loss, {"a0_loss": loss})
